# baseline (speedup 1.0000x reference)
import functools

import jax
import jax.numpy as jnp
from jax import lax
from jax.experimental import pallas as pl
from jax.experimental.pallas import tpu as pltpu
from jax.experimental.pallas import tpu_sc as plsc

BATCH = 16384
SEQ = 10
EMBED_DIM = 64
VOCAB = 1000

_NC = 2
_NS = 16
_NW = _NC * _NS
_ROWS = BATCH * SEQ
_RPW = _ROWS // _NW
_CHUNK = 128
_NCH = _RPW // _CHUNK
_NBUF = 4
_L = 16


@functools.partial(
    pl.kernel,
    mesh=plsc.VectorSubcoreMesh(core_axis_name="c", subcore_axis_name="s"),
    out_type=jax.ShapeDtypeStruct((_ROWS * EMBED_DIM,), jnp.float32),
    scratch_types=[
        pltpu.VMEM((VOCAB * EMBED_DIM,), jnp.float32),
        pltpu.VMEM((_RPW,), jnp.int32),
        pltpu.VMEM((_NBUF, _CHUNK * EMBED_DIM), jnp.float32),
        pltpu.SemaphoreType.DMA((_NBUF,)),
    ],
    compiler_params=pltpu.CompilerParams(
        use_tc_tiling_on_sc=False, needs_layout_passes=False
    ),
)
def _embedding_rows(idx_hbm, table_hbm, out_hbm, table_v, idx_v, stage_v,
                    wsem):
    wid = lax.axis_index("s") * _NC + lax.axis_index("c")
    r0 = wid * _RPW

    pltpu.sync_copy(table_hbm, table_v)
    pltpu.sync_copy(idx_hbm.at[pl.ds(r0, _RPW)], idx_v)

    siota = lax.iota(jnp.int32, _L) * EMBED_DIM

    def fill(c, buf):
        for g in range(_CHUNK // _L):
            iv = idx_v[pl.ds(c * _CHUNK + g * _L, _L)]
            src = iv * EMBED_DIM
            dst = siota + g * (_L * EMBED_DIM)
            for _ in range(EMBED_DIM):
                vals = plsc.load_gather(table_v, [src])
                plsc.store_scatter(stage_v.at[buf], [dst], vals)
                src = src + 1
                dst = dst + 1

    def start_write(c, buf):
        pltpu.async_copy(
            stage_v.at[buf],
            out_hbm.at[pl.ds((r0 + c * _CHUNK) * EMBED_DIM,
                             _CHUNK * EMBED_DIM)],
            wsem.at[buf])

    def wait_write(c, buf):
        pltpu.make_async_copy(
            stage_v.at[buf],
            out_hbm.at[pl.ds((r0 + c * _CHUNK) * EMBED_DIM,
                             _CHUNK * EMBED_DIM)],
            wsem.at[buf]).wait()

    @pl.loop(0, _NCH)
    def _chunk(c):
        buf = c & (_NBUF - 1)

        @pl.when(c >= _NBUF)
        def _():
            wait_write(c - _NBUF, buf)

        fill(c, buf)
        start_write(c, buf)

    for c in range(_NCH - _NBUF, _NCH):
        wait_write(c, c % _NBUF)


def kernel(inputs, table):
    idx1 = inputs.reshape(_ROWS)
    table1 = table.reshape(VOCAB * EMBED_DIM)
    out = _embedding_rows(idx1, table1)
    return out.reshape(BATCH, SEQ, EMBED_DIM)


# speedup vs baseline: 2.7088x; 2.7088x over previous
import functools

import jax
import jax.numpy as jnp
from jax import lax
from jax.experimental import pallas as pl
from jax.experimental.pallas import tpu as pltpu
from jax.experimental.pallas import tpu_sc as plsc

BATCH = 16384
SEQ = 10
EMBED_DIM = 64
VOCAB = 1000

_NC = 2
_NS = 16
_NW = _NC * _NS
_ROWS = BATCH * SEQ
_RPW = _ROWS // _NW
_CHUNK = 128
_NCH = _RPW // _CHUNK
_NPAIR = _NCH // 2


@functools.partial(
    pl.kernel,
    mesh=plsc.VectorSubcoreMesh(core_axis_name="c", subcore_axis_name="s"),
    out_type=jax.ShapeDtypeStruct((_ROWS, EMBED_DIM), jnp.float32),
    scratch_types=[
        pltpu.VMEM((_NCH, _CHUNK), jnp.int32),
        pltpu.VMEM((4, _CHUNK, EMBED_DIM), jnp.float32),
        pltpu.VMEM_SHARED((VOCAB, EMBED_DIM), jnp.float32),
        pltpu.SemaphoreType.DMA((2,)),
        pltpu.SemaphoreType.DMA((2,)),
        pltpu.SemaphoreType.DMA((4,)),
    ],
    compiler_params=pltpu.CompilerParams(use_tc_tiling_on_sc=False),
)
def _embedding_rows(idx_hbm, table_hbm, out_hbm, idx_v, rows_v, table_v,
                    ssem, hsem, wsem):
    wid = lax.axis_index("s") * _NC + lax.axis_index("c")
    r0 = wid * _RPW

    @pl.when(lax.axis_index("s") == 0)
    def _():
        pltpu.sync_copy(table_hbm, table_v)

    pltpu.sync_copy(idx_hbm.at[pl.ds(wid * _NCH, _NCH)], idx_v)
    plsc.subcore_barrier()

    def start_gathers(g, pb):
        pltpu.async_copy(table_v.at[idx_v.at[2 * g]], rows_v.at[2 * pb],
                         ssem.at[pb])
        pltpu.async_copy(table_hbm.at[idx_v.at[2 * g + 1]],
                         rows_v.at[2 * pb + 1], hsem.at[pb])

    def wait_gathers(g, pb):
        pltpu.make_async_copy(table_v.at[idx_v.at[2 * g]], rows_v.at[2 * pb],
                              ssem.at[pb]).wait()
        pltpu.make_async_copy(table_hbm.at[idx_v.at[2 * g + 1]],
                              rows_v.at[2 * pb + 1], hsem.at[pb]).wait()

    def start_writes(g, pb):
        for k in range(2):
            pltpu.async_copy(
                rows_v.at[2 * pb + k],
                out_hbm.at[pl.ds(r0 + (2 * g + k) * _CHUNK, _CHUNK)],
                wsem.at[2 * pb + k])

    def wait_writes(g, pb):
        for k in range(2):
            pltpu.make_async_copy(
                rows_v.at[2 * pb + k],
                out_hbm.at[pl.ds(r0 + (2 * g + k) * _CHUNK, _CHUNK)],
                wsem.at[2 * pb + k]).wait()

    @pl.loop(0, _NPAIR)
    def _pair(g):
        pb = g & 1

        @pl.when(g >= 2)
        def _():
            wait_writes(g - 2, pb)

        start_gathers(g, pb)

        @pl.when(g >= 1)
        def _():
            wait_gathers(g - 1, 1 - pb)
            start_writes(g - 1, 1 - pb)

    wait_gathers(_NPAIR - 1, (_NPAIR - 1) & 1)
    start_writes(_NPAIR - 1, (_NPAIR - 1) & 1)
    for g in (_NPAIR - 2, _NPAIR - 1):
        wait_writes(g, g & 1)


def kernel(inputs, table):
    idx2 = inputs.reshape(_NW * _NCH, _CHUNK)
    out = _embedding_rows(idx2, table)
    return out.reshape(BATCH, SEQ, EMBED_DIM)
